# Initial kernel scaffold; baseline (speedup 1.0000x reference)
import jax
import jax.numpy as jnp
from jax.experimental import pallas as pl


def kernel(x, arch_params, codes, table, codebooks, linear_w, linear_b):
    raise NotImplementedError("write your pallas kernel here")


# speedup vs baseline: 1.2007x; 1.2007x over previous
import numpy as np
import jax
import jax.numpy as jnp
from jax.experimental import pallas as pl

_K_SPACE = [1, 64, 128, 256, 512, 1024, 2048]
_N_FIELDS = 26
_FIELD_DIM = 4000
_DIM = 128
_M = 16
_QS = _DIM // _M
_TEMPERATURE = 1.0
_THRESHOLD = 500
_BLK = 128


def _prior_mask_np():
    mask = np.full((_N_FIELDS, len(_K_SPACE)), -1e9, dtype=np.float32)
    for i in range(_N_FIELDS):
        if _FIELD_DIM < _THRESHOLD:
            mask[i, 0] = 0.0
        for k in range(1, len(_K_SPACE)):
            if _K_SPACE[k] * 2.5 > _FIELD_DIM:
                break
            mask[i, k] = 0.0
    return mask


_MASK = _prior_mask_np()
_ACTIVE = [k for k in range(len(_K_SPACE)) if (_MASK[:, k] == 0.0).any()]


def _fused_kernel(ap_ref, lw_ref, lb_ref, *refs):
    e_refs = refs[:-1]
    out_ref = refs[-1]
    z = ap_ref[...]
    z = z - jnp.max(z, axis=1, keepdims=True)
    ez = jnp.exp(z / _TEMPERATURE)
    prob = ez / jnp.sum(ez, axis=1, keepdims=True)
    acc = prob[None, :, _ACTIVE[0], None] * e_refs[0][...]
    for j, k in enumerate(_ACTIVE[1:], start=1):
        acc = acc + prob[None, :, k, None] * e_refs[j][...]
    s = jnp.sum(acc, axis=1)
    sq = jnp.sum(acc * acc, axis=1)
    fm = 0.5 * jnp.sum(s * s - sq, axis=1)
    lin = jnp.sum(lw_ref[...], axis=1) + lb_ref[0, 0]
    out_ref[...] = (fm + lin)[:, None]


@jax.jit
def kernel(x, arch_params, codes, table, codebooks, linear_w, linear_b):
    batch = x.shape[0]
    offsets = (jnp.arange(_N_FIELDS, dtype=jnp.int32) * _FIELD_DIM)[None, :]
    idx = x + offsets
    m_idx = jnp.arange(_M)[None, None, :]
    embs = []
    for k in _ACTIVE:
        ck = jnp.take(codes[k], idx, axis=0) % _K_SPACE[k]
        ek = codebooks[k][m_idx, ck].reshape(batch, _N_FIELDS, _DIM)
        embs.append(ek)
    lw = jnp.take(linear_w, idx, axis=0)
    lb = linear_b.reshape(1, 1)
    nblk = batch // _BLK

    e_spec = pl.BlockSpec((_BLK, _N_FIELDS, _DIM), lambda i: (i, 0, 0))
    out = pl.pallas_call(
        _fused_kernel,
        grid=(nblk,),
        in_specs=[
            pl.BlockSpec((_N_FIELDS, len(_K_SPACE)), lambda i: (0, 0)),
            pl.BlockSpec((_BLK, _N_FIELDS), lambda i: (i, 0)),
            pl.BlockSpec((1, 1), lambda i: (0, 0)),
        ] + [e_spec] * len(_ACTIVE),
        out_specs=pl.BlockSpec((_BLK, 1), lambda i: (i, 0)),
        out_shape=jax.ShapeDtypeStruct((batch, 1), jnp.float32),
    )(arch_params + jnp.asarray(_MASK), lw, lb, *embs)
    return out.reshape(batch)
